# Initial kernel scaffold; baseline (speedup 1.0000x reference)
import jax
import jax.numpy as jnp
from jax.experimental import pallas as pl


def kernel(x, edge_index, batch, W1, b1, W2, b2):
    raise NotImplementedError("write your pallas kernel here")


# speedup vs baseline: 559.0741x; 559.0741x over previous
import functools

import jax
import jax.numpy as jnp
from jax import lax
from jax.experimental import pallas as pl
from jax.experimental.pallas import tpu as pltpu
from jax.experimental.pallas import tpu_sc as plsc

N_NODES = 10000
N_PAD = 10240
D_FEAT = 128
HID = 16
NUM_CLASSES = 10
NUM_GRAPHS = 16
NC = 2
NS = 16
NW = NC * NS
CHUNK = 128
ROWS_PER_TILE = N_PAD // NS


def _wid(c, s):
    return s * NC + c


def _make_deg_kernel(tpw):
    mesh = plsc.VectorSubcoreMesh(core_axis_name="c", subcore_axis_name="s")

    @functools.partial(
        pl.kernel,
        out_type=jax.ShapeDtypeStruct((NW, N_PAD), jnp.float32),
        mesh=mesh,
        scratch_types=[
            pltpu.VMEM((tpw,), jnp.int32),
            pltpu.VMEM((N_PAD,), jnp.float32),
        ],
        compiler_params=pltpu.CompilerParams(needs_layout_passes=False),
    )
    def deg_kernel(dst_hbm, out_hbm, dst_v, hist_v):
        w = _wid(lax.axis_index("c"), lax.axis_index("s"))
        pltpu.sync_copy(dst_hbm.at[w], dst_v)
        zeros16 = jnp.zeros((16,), jnp.float32)

        def zbody(i, _):
            hist_v[pl.ds(i * 16, 16)] = zeros16
            return _

        lax.fori_loop(0, N_PAD // 16, zbody, None)
        ones16 = jnp.ones((16,), jnp.float32)

        def cbody(k, _):
            idx = dst_v[pl.ds(k * 16, 16)]
            plsc.addupdate_scatter(hist_v, [idx], ones16)
            return _

        lax.fori_loop(0, tpw // 16, cbody, None)
        pltpu.sync_copy(hist_v, out_hbm.at[w])

    return deg_kernel


def _make_agg_kernel(tpw):
    nch = tpw // CHUNK
    mesh = plsc.VectorSubcoreMesh(core_axis_name="c", subcore_axis_name="s")

    @functools.partial(
        pl.kernel,
        out_type=jax.ShapeDtypeStruct((NC, N_PAD, HID), jnp.float32),
        mesh=mesh,
        scratch_types=[
            pltpu.VMEM((tpw,), jnp.int32),
            pltpu.VMEM((tpw,), jnp.int32),
            pltpu.VMEM((CHUNK,), jnp.int32),
            pltpu.VMEM((CHUNK, HID), jnp.float32),
            pltpu.VMEM_SHARED((N_PAD, HID), jnp.float32),
            pltpu.SemaphoreType.DMA,
        ],
        compiler_params=pltpu.CompilerParams(
            needs_layout_passes=False, use_tc_tiling_on_sc=False),
    )
    def agg_kernel(g_hbm, src_hbm, dst_hbm, out_hbm,
                   src_v, dst_v, idx_v, rows_v, acc_sh, sem):
        c = lax.axis_index("c")
        s = lax.axis_index("s")
        w = _wid(c, s)
        pltpu.sync_copy(src_hbm.at[w], src_v)
        pltpu.sync_copy(dst_hbm.at[w], dst_v)

        zeros16 = jnp.zeros((16,), jnp.float32)

        def zbody(i, _):
            rows_v[i, :] = zeros16
            return _

        lax.fori_loop(0, CHUNK, zbody, None)
        base = s * ROWS_PER_TILE
        for k in range(ROWS_PER_TILE // CHUNK):
            pltpu.sync_copy(rows_v, acc_sh.at[pl.ds(base + k * CHUNK, CHUNK)])
        plsc.subcore_barrier()

        def chunk(j, _):
            for k in range(CHUNK // 16):
                idx_v[pl.ds(k * 16, 16)] = dst_v[pl.ds(j * CHUNK + k * 16, 16)]
            gat = pltpu.async_copy(
                g_hbm.at[src_v.at[pl.ds(j * CHUNK, CHUNK)]], rows_v, sem)
            gat.wait()
            pltpu.sync_copy(rows_v, acc_sh.at[idx_v], add=True)
            return _

        lax.fori_loop(0, nch, chunk, None)
        plsc.subcore_barrier()
        pltpu.sync_copy(acc_sh.at[pl.ds(base, ROWS_PER_TILE)],
                        out_hbm.at[c, pl.ds(base, ROWS_PER_TILE)])

    return agg_kernel


def _a_body(x_ref, w_ref, deg_ref, g_ref):
    deg_in = jnp.sum(deg_ref[...], axis=0)
    dinv = lax.rsqrt(deg_in + 1.0)
    h = jnp.dot(x_ref[...], w_ref[...], preferred_element_type=jnp.float32)
    g_ref[...] = h * dinv[:, None]


def _c_body(acc_ref, g1_ref, deg_ref, w_ref, b_ref, g2_ref):
    deg_in = jnp.sum(deg_ref[...], axis=0)
    dinv = lax.rsqrt(deg_in + 1.0)
    srow = acc_ref[0] + acc_ref[1] + g1_ref[...]
    out1 = jnp.maximum(dinv[:, None] * srow + b_ref[...], 0.0)
    h2 = jnp.dot(out1, w_ref[...], preferred_element_type=jnp.float32)
    g2_ref[...] = h2 * dinv[:, None]


def _e_body(acc_ref, g2_ref, deg_ref, b_ref, batch_ref, out_ref):
    deg_in = jnp.sum(deg_ref[...], axis=0)
    dinv = lax.rsqrt(deg_in + 1.0)
    srow = acc_ref[0] + acc_ref[1] + g2_ref[...]
    out2 = dinv[:, None] * srow + b_ref[...]
    gid = lax.broadcasted_iota(jnp.int32, (NUM_GRAPHS, N_PAD), 0)
    oh = (gid == batch_ref[...]).astype(jnp.float32)
    sums = jnp.dot(oh, out2, preferred_element_type=jnp.float32)
    counts = jnp.sum(oh, axis=1)
    pooled = sums / jnp.maximum(counts, 1.0)[:, None]
    logits = pooled[:, :NUM_CLASSES]
    m = jnp.max(logits, axis=1, keepdims=True)
    shifted = logits - m
    out_ref[...] = shifted - jnp.log(
        jnp.sum(jnp.exp(shifted), axis=1, keepdims=True))


_NB = 1024


def _tc_scale1(xp, W1, deg32):
    return pl.pallas_call(
        _a_body,
        grid=(N_PAD // _NB,),
        in_specs=[
            pl.BlockSpec((_NB, D_FEAT), lambda i: (i, 0)),
            pl.BlockSpec((D_FEAT, HID), lambda i: (0, 0)),
            pl.BlockSpec((NW, _NB), lambda i: (0, i)),
        ],
        out_specs=pl.BlockSpec((_NB, HID), lambda i: (i, 0)),
        out_shape=jax.ShapeDtypeStruct((N_PAD, HID), jnp.float32),
    )(xp, W1, deg32)


def _tc_scale2(acc1, g1, deg32, W2p, b1r):
    return pl.pallas_call(
        _c_body,
        grid=(N_PAD // _NB,),
        in_specs=[
            pl.BlockSpec((NC, _NB, HID), lambda i: (0, i, 0)),
            pl.BlockSpec((_NB, HID), lambda i: (i, 0)),
            pl.BlockSpec((NW, _NB), lambda i: (0, i)),
            pl.BlockSpec((HID, HID), lambda i: (0, 0)),
            pl.BlockSpec((1, HID), lambda i: (0, 0)),
        ],
        out_specs=pl.BlockSpec((_NB, HID), lambda i: (i, 0)),
        out_shape=jax.ShapeDtypeStruct((N_PAD, HID), jnp.float32),
    )(acc1, g1, deg32, W2p, b1r)


def _tc_pool(acc2, g2, deg32, b2r, batchp):
    return pl.pallas_call(
        _e_body,
        in_specs=[
            pl.BlockSpec(acc2.shape, lambda: (0, 0, 0)),
            pl.BlockSpec(g2.shape, lambda: (0, 0)),
            pl.BlockSpec(deg32.shape, lambda: (0, 0)),
            pl.BlockSpec(b2r.shape, lambda: (0, 0)),
            pl.BlockSpec(batchp.shape, lambda: (0, 0)),
        ],
        out_specs=pl.BlockSpec((NUM_GRAPHS, NUM_CLASSES), lambda: (0, 0)),
        out_shape=jax.ShapeDtypeStruct((NUM_GRAPHS, NUM_CLASSES), jnp.float32),
    )(acc2, g2, deg32, b2r, batchp)


def kernel(x, edge_index, batch, W1, b1, W2, b2):
    args = (x.astype(jnp.float32),
            edge_index.astype(jnp.int32),
            batch.astype(jnp.int32),
            W1.astype(jnp.float32), b1.astype(jnp.float32),
            W2.astype(jnp.float32), b2.astype(jnp.float32))
    with jax.enable_x64(False):
        out = _impl(*args)
    return out.astype(jnp.float64)


def _impl(x, edge_index, batch, W1, b1, W2, b2):
    n_edges = edge_index.shape[1]
    e_pad = -(-n_edges // (NW * CHUNK)) * (NW * CHUNK)
    tpw = e_pad // NW

    src = edge_index[0]
    dst = edge_index[1]
    pad_idx = (jnp.arange(e_pad - n_edges, dtype=jnp.int32)
               % (N_PAD - N_NODES)) + N_NODES
    srcp = jnp.concatenate([src, pad_idx]).reshape(NW, tpw)
    dstp = jnp.concatenate([dst, pad_idx]).reshape(NW, tpw)

    xp = jnp.pad(x.astype(jnp.float32), ((0, N_PAD - N_NODES), (0, 0)))
    W1f = W1.astype(jnp.float32)
    W2p = jnp.pad(W2.astype(jnp.float32), ((0, 0), (0, HID - NUM_CLASSES)))
    b1r = b1.astype(jnp.float32).reshape(1, HID)
    b2r = jnp.pad(b2.astype(jnp.float32), (0, HID - NUM_CLASSES)).reshape(1, HID)
    batchp = jnp.pad(batch.astype(jnp.int32), (0, N_PAD - N_NODES),
                     constant_values=NUM_GRAPHS).reshape(1, N_PAD)

    deg32 = _make_deg_kernel(tpw)(dstp)
    g1 = _tc_scale1(xp, W1f, deg32)
    agg = _make_agg_kernel(tpw)
    acc1 = agg(g1, srcp, dstp)
    g2 = _tc_scale2(acc1, g1, deg32, W2p, b1r)
    acc2 = agg(g2, srcp, dstp)
    return _tc_pool(acc2, g2, deg32, b2r, batchp)


# speedup vs baseline: 931.6116x; 1.6663x over previous
import functools

import jax
import jax.numpy as jnp
from jax import lax
from jax.experimental import pallas as pl
from jax.experimental.pallas import tpu as pltpu
from jax.experimental.pallas import tpu_sc as plsc

N_NODES = 10000
N_PAD = 10240
D_FEAT = 128
HID = 16
NUM_CLASSES = 10
NUM_GRAPHS = 16
NC = 2
NS = 16
NW = NC * NS
CHUNK = 128
ROWS_PER_TILE = N_PAD // NS


def _wid(c, s):
    return s * NC + c


def _make_deg_kernel(tpw):
    mesh = plsc.VectorSubcoreMesh(core_axis_name="c", subcore_axis_name="s")

    @functools.partial(
        pl.kernel,
        out_type=jax.ShapeDtypeStruct((NW, N_PAD), jnp.float32),
        mesh=mesh,
        scratch_types=[
            pltpu.VMEM((tpw,), jnp.int32),
            pltpu.VMEM((N_PAD,), jnp.float32),
        ],
        compiler_params=pltpu.CompilerParams(needs_layout_passes=False),
    )
    def deg_kernel(dst_hbm, out_hbm, dst_v, hist_v):
        w = _wid(lax.axis_index("c"), lax.axis_index("s"))
        pltpu.sync_copy(dst_hbm.at[w], dst_v)
        zeros16 = jnp.zeros((16,), jnp.float32)

        def zbody(i, _):
            hist_v[pl.ds(i * 16, 16)] = zeros16
            return _

        lax.fori_loop(0, N_PAD // 16, zbody, None)
        ones16 = jnp.ones((16,), jnp.float32)

        def cbody(k, _):
            idx = dst_v[pl.ds(k * 16, 16)]
            plsc.addupdate_scatter(hist_v, [idx], ones16)
            return _

        lax.fori_loop(0, tpw // 16, cbody, None)
        pltpu.sync_copy(hist_v, out_hbm.at[w])

    return deg_kernel


NBUF = 4


def _make_agg_kernel(tpw):
    nch = tpw // CHUNK
    assert nch % NBUF == 0
    n_grp = nch // NBUF
    mesh = plsc.VectorSubcoreMesh(core_axis_name="c", subcore_axis_name="s")

    @functools.partial(
        pl.kernel,
        out_type=jax.ShapeDtypeStruct((NC, N_PAD, HID), jnp.float32),
        mesh=mesh,
        scratch_types=(
            [pltpu.VMEM((tpw,), jnp.int32)] * 2
            + [pltpu.VMEM((CHUNK,), jnp.int32)] * NBUF
            + [pltpu.VMEM((CHUNK, HID), jnp.float32)] * NBUF
            + [pltpu.VMEM_SHARED((N_PAD, HID), jnp.float32)]
            + [pltpu.SemaphoreType.DMA] * NBUF
        ),
        compiler_params=pltpu.CompilerParams(
            needs_layout_passes=False, use_tc_tiling_on_sc=False),
    )
    def agg_kernel(g_hbm, src_hbm, dst_hbm, out_hbm, src_v, dst_v, *rest):
        idx_v = rest[:NBUF]
        rows_v = rest[NBUF:2 * NBUF]
        acc_sh = rest[2 * NBUF]
        gsem = rest[2 * NBUF + 1:]
        c = lax.axis_index("c")
        s = lax.axis_index("s")
        w = _wid(c, s)
        pltpu.sync_copy(src_hbm.at[w], src_v)
        pltpu.sync_copy(dst_hbm.at[w], dst_v)

        zeros16 = jnp.zeros((16,), jnp.float32)

        def zbody(i, _):
            rows_v[0][i, :] = zeros16
            return _

        lax.fori_loop(0, CHUNK, zbody, None)
        base = s * ROWS_PER_TILE
        for k in range(ROWS_PER_TILE // CHUNK):
            pltpu.sync_copy(rows_v[0], acc_sh.at[pl.ds(base + k * CHUNK, CHUNK)])
        plsc.subcore_barrier()

        def gather_start(j, b):
            pltpu.async_copy(
                g_hbm.at[src_v.at[pl.ds(j * CHUNK, CHUNK)]], rows_v[b], gsem[b])

        def gather_wait(j, b):
            pltpu.make_async_copy(
                g_hbm.at[src_v.at[pl.ds(j * CHUNK, CHUNK)]], rows_v[b],
                gsem[b]).wait()

        def consume(j, b, refill):
            for k in range(CHUNK // 16):
                idx_v[b][pl.ds(k * 16, 16)] = dst_v[pl.ds(j * CHUNK + k * 16, 16)]
            gather_wait(j, b)
            pltpu.sync_copy(rows_v[b], acc_sh.at[idx_v[b]], add=True)
            if refill:
                gather_start(j + NBUF, b)

        for b in range(NBUF):
            gather_start(jnp.int32(b), b)

        def group(jo, _):
            for b in range(NBUF):
                consume(jo * NBUF + b, b, refill=True)
            return _

        lax.fori_loop(0, n_grp - 1, group, None)
        for b in range(NBUF):
            consume(jnp.int32((n_grp - 1) * NBUF + b), b, refill=False)

        plsc.subcore_barrier()
        pltpu.sync_copy(acc_sh.at[pl.ds(base, ROWS_PER_TILE)],
                        out_hbm.at[c, pl.ds(base, ROWS_PER_TILE)])

    return agg_kernel


def _a_body(x_ref, w_ref, deg_ref, g_ref):
    deg_in = jnp.sum(deg_ref[...], axis=0)
    dinv = lax.rsqrt(deg_in + 1.0)
    h = jnp.dot(x_ref[...], w_ref[...], preferred_element_type=jnp.float32)
    g_ref[...] = h * dinv[:, None]


def _c_body(acc_ref, g1_ref, deg_ref, w_ref, b_ref, g2_ref):
    deg_in = jnp.sum(deg_ref[...], axis=0)
    dinv = lax.rsqrt(deg_in + 1.0)
    srow = acc_ref[0] + acc_ref[1] + g1_ref[...]
    out1 = jnp.maximum(dinv[:, None] * srow + b_ref[...], 0.0)
    h2 = jnp.dot(out1, w_ref[...], preferred_element_type=jnp.float32)
    g2_ref[...] = h2 * dinv[:, None]


def _e_body(acc_ref, g2_ref, deg_ref, b_ref, batch_ref, out_ref):
    deg_in = jnp.sum(deg_ref[...], axis=0)
    dinv = lax.rsqrt(deg_in + 1.0)
    srow = acc_ref[0] + acc_ref[1] + g2_ref[...]
    out2 = dinv[:, None] * srow + b_ref[...]
    gid = lax.broadcasted_iota(jnp.int32, (NUM_GRAPHS, N_PAD), 0)
    oh = (gid == batch_ref[...]).astype(jnp.float32)
    sums = jnp.dot(oh, out2, preferred_element_type=jnp.float32)
    counts = jnp.sum(oh, axis=1)
    pooled = sums / jnp.maximum(counts, 1.0)[:, None]
    logits = pooled[:, :NUM_CLASSES]
    m = jnp.max(logits, axis=1, keepdims=True)
    shifted = logits - m
    out_ref[...] = shifted - jnp.log(
        jnp.sum(jnp.exp(shifted), axis=1, keepdims=True))


_NB = 1024


def _tc_scale1(xp, W1, deg32):
    return pl.pallas_call(
        _a_body,
        grid=(N_PAD // _NB,),
        in_specs=[
            pl.BlockSpec((_NB, D_FEAT), lambda i: (i, 0)),
            pl.BlockSpec((D_FEAT, HID), lambda i: (0, 0)),
            pl.BlockSpec((NW, _NB), lambda i: (0, i)),
        ],
        out_specs=pl.BlockSpec((_NB, HID), lambda i: (i, 0)),
        out_shape=jax.ShapeDtypeStruct((N_PAD, HID), jnp.float32),
    )(xp, W1, deg32)


def _tc_scale2(acc1, g1, deg32, W2p, b1r):
    return pl.pallas_call(
        _c_body,
        grid=(N_PAD // _NB,),
        in_specs=[
            pl.BlockSpec((NC, _NB, HID), lambda i: (0, i, 0)),
            pl.BlockSpec((_NB, HID), lambda i: (i, 0)),
            pl.BlockSpec((NW, _NB), lambda i: (0, i)),
            pl.BlockSpec((HID, HID), lambda i: (0, 0)),
            pl.BlockSpec((1, HID), lambda i: (0, 0)),
        ],
        out_specs=pl.BlockSpec((_NB, HID), lambda i: (i, 0)),
        out_shape=jax.ShapeDtypeStruct((N_PAD, HID), jnp.float32),
    )(acc1, g1, deg32, W2p, b1r)


def _tc_pool(acc2, g2, deg32, b2r, batchp):
    return pl.pallas_call(
        _e_body,
        in_specs=[
            pl.BlockSpec(acc2.shape, lambda: (0, 0, 0)),
            pl.BlockSpec(g2.shape, lambda: (0, 0)),
            pl.BlockSpec(deg32.shape, lambda: (0, 0)),
            pl.BlockSpec(b2r.shape, lambda: (0, 0)),
            pl.BlockSpec(batchp.shape, lambda: (0, 0)),
        ],
        out_specs=pl.BlockSpec((NUM_GRAPHS, NUM_CLASSES), lambda: (0, 0)),
        out_shape=jax.ShapeDtypeStruct((NUM_GRAPHS, NUM_CLASSES), jnp.float32),
    )(acc2, g2, deg32, b2r, batchp)


def kernel(x, edge_index, batch, W1, b1, W2, b2):
    args = (x.astype(jnp.float32),
            edge_index.astype(jnp.int32),
            batch.astype(jnp.int32),
            W1.astype(jnp.float32), b1.astype(jnp.float32),
            W2.astype(jnp.float32), b2.astype(jnp.float32))
    with jax.enable_x64(False):
        out = _impl(*args)
    return out.astype(jnp.float64)


def _impl(x, edge_index, batch, W1, b1, W2, b2):
    n_edges = edge_index.shape[1]
    e_pad = -(-n_edges // (NW * CHUNK * NBUF)) * (NW * CHUNK * NBUF)
    tpw = e_pad // NW

    src = edge_index[0]
    dst = edge_index[1]
    pad_idx = (jnp.arange(e_pad - n_edges, dtype=jnp.int32)
               % (N_PAD - N_NODES)) + N_NODES
    srcp = jnp.concatenate([src, pad_idx]).reshape(NW, tpw)
    dstp = jnp.concatenate([dst, pad_idx]).reshape(NW, tpw)

    xp = jnp.pad(x.astype(jnp.float32), ((0, N_PAD - N_NODES), (0, 0)))
    W1f = W1.astype(jnp.float32)
    W2p = jnp.pad(W2.astype(jnp.float32), ((0, 0), (0, HID - NUM_CLASSES)))
    b1r = b1.astype(jnp.float32).reshape(1, HID)
    b2r = jnp.pad(b2.astype(jnp.float32), (0, HID - NUM_CLASSES)).reshape(1, HID)
    batchp = jnp.pad(batch.astype(jnp.int32), (0, N_PAD - N_NODES),
                     constant_values=NUM_GRAPHS).reshape(1, N_PAD)

    deg32 = _make_deg_kernel(tpw)(dstp)
    g1 = _tc_scale1(xp, W1f, deg32)
    agg = _make_agg_kernel(tpw)
    acc1 = agg(g1, srcp, dstp)
    g2 = _tc_scale2(acc1, g1, deg32, W2p, b1r)
    acc2 = agg(g2, srcp, dstp)
    return _tc_pool(acc2, g2, deg32, b2r, batchp)
